# baseline (speedup 1.0000x reference)
import functools

import jax
import jax.numpy as jnp
import numpy as np
from jax.experimental import pallas as pl
from jax.experimental.pallas import tpu as pltpu

_POINT_NUM = 32768
_F = 200000
_FT = 512
_FPAD = 200192
_NFT = _FPAD // _FT
_PBLK = 2048
_PT = 32

_U = np.uint32
_TINY = np.float32(1.1754943508222875e-38)


def _rotl(x, r):
    return jax.lax.shift_left(x, _U(r)) | jax.lax.shift_right_logical(x, _U(32 - r))


def _threefry(k1, k2, x0, x1):
    ks2 = k1 ^ k2 ^ _U(0x1BD11BDA)
    rot = ((13, 15, 26, 6), (17, 29, 16, 24))
    sched = ((0, k2, ks2, 1), (1, ks2, k1, 2), (0, k1, k2, 3),
             (1, k2, ks2, 4), (0, ks2, k1, 5))
    x0 = x0 + k1
    x1 = x1 + k2
    for rsel, ka, kb, inc in sched:
        for r in rot[rsel]:
            x0 = x0 + x1
            x1 = _rotl(x1, r)
            x1 = x0 ^ x1
        x0 = x0 + ka
        x1 = x1 + (kb + _U(inc))
    return x0, x1


def _bits_to_unit(bits):
    fb = jax.lax.shift_right_logical(bits, _U(9)) | _U(0x3F800000)
    return jax.lax.bitcast_convert_type(fb, jnp.float32) - jnp.float32(1.0)


def _argmax_kernel(keys_ref, logits_ref, oidx_ref):
    b = pl.program_id(0)
    pblk = pl.program_id(1)
    k1 = keys_ref[b, 0].astype(jnp.uint32)
    k2 = keys_ref[b, 1].astype(jnp.uint32)

    f_lane = jax.lax.broadcasted_iota(jnp.int32, (_PT, _FT), 1)
    p_sub = jax.lax.broadcasted_iota(jnp.int32, (_PT, _FT), 0)
    p_cross = (1 << 32) // _F + 1

    def pg_body(pg, _):
        p_abs = pblk * _PBLK + pg * _PT + p_sub
        pf_lo = (p_abs * _F).astype(jnp.uint32)
        hi1 = (p_abs >= p_cross).astype(jnp.uint32)

        def ft_body(ft, carry):
            runval, runidx = carry
            f_abs = ft * _FT + f_lane
            n_lo = pf_lo + f_abs.astype(jnp.uint32)
            n_hi = hi1 + (n_lo < pf_lo).astype(jnp.uint32)
            o0, o1 = _threefry(k1, k2, n_hi, n_lo)
            bits = o0 ^ o1
            u = jnp.maximum(_bits_to_unit(bits), _TINY)
            g = -jnp.log(-jnp.log(u))
            lg = logits_ref[0, 0, pl.ds(ft * _FT, _FT)].reshape(1, _FT)
            v = g + lg
            upd = v > runval
            return (jnp.where(upd, v, runval), jnp.where(upd, f_abs, runidx))

        init = (jnp.full((_PT, _FT), -3.4e38, jnp.float32),
                jnp.zeros((_PT, _FT), jnp.int32))
        runval, runidx = jax.lax.fori_loop(0, _NFT, ft_body, init, unroll=17)
        rowmax = jnp.max(runval, axis=1, keepdims=True)
        masked = jnp.where(runval == rowmax, runidx, jnp.int32(0x7FFFFFFF))
        rowidx = jnp.min(masked, axis=1, keepdims=True)
        oidx_ref[0, pl.ds(pg * _PT, _PT), :] = rowidx
        return 0

    jax.lax.fori_loop(0, _PBLK // _PT, pg_body, 0)


def _categorical_argmax(keys_i32, logits_padded):
    B = logits_padded.shape[0]
    out = pl.pallas_call(
        _argmax_kernel,
        grid=(B, _POINT_NUM // _PBLK),
        in_specs=[
            pl.BlockSpec(memory_space=pltpu.SMEM),
            pl.BlockSpec((1, 1, _FPAD), lambda b, p: (b, 0, 0)),
        ],
        out_specs=pl.BlockSpec((1, _PBLK, 1), lambda b, p: (b, p, 0)),
        out_shape=jax.ShapeDtypeStruct((B, _POINT_NUM, 1), jnp.int32),
    )(keys_i32, logits_padded.reshape(B, 1, _FPAD))
    return out[:, :, 0]


def _combine_kernel(a_ref, b_ref, c_ref, w1_ref, w2_ref, w3_ref, o_ref):
    o_ref[...] = (a_ref[...] * w1_ref[...] + b_ref[...] * w2_ref[...]
                  + c_ref[...] * w3_ref[...])


def _sample_shard(vb, faces_batch, ki_data, ke1_data, ke2_data):
    B = vb.shape[0]
    ke1 = jax.random.wrap_key_data(
        jax.lax.bitcast_convert_type(ke1_data, jnp.uint32), impl="threefry2x32")
    ke2 = jax.random.wrap_key_data(
        jax.lax.bitcast_convert_type(ke2_data, jnp.uint32), impl="threefry2x32")

    def mesh_logits(vertices, faces):
        triangles = jnp.take(vertices, faces, axis=0)
        vec1 = triangles[:, 1, :] - triangles[:, 0, :]
        vec2 = triangles[:, 2, :] - triangles[:, 0, :]
        areas = jnp.linalg.norm(jnp.cross(vec1, vec2, axis=-1), axis=1) / 2.0
        return triangles, jnp.log(areas + 1e-12)

    triangles, logits = jax.vmap(mesh_logits)(vb, faces_batch)
    logits_padded = jnp.concatenate(
        [logits, jnp.full((B, _FPAD - _F), -1e30, jnp.float32)], axis=1)

    idx = _categorical_argmax(ki_data, logits_padded)

    st = jax.vmap(lambda t, i: jnp.take(t, i, axis=0))(triangles, idx)

    eps1 = jax.vmap(lambda k: jax.random.uniform(
        k, (_POINT_NUM,), dtype=jnp.float32))(ke1)
    eps2 = jax.vmap(lambda k: jax.random.uniform(
        k, (_POINT_NUM,), dtype=jnp.float32))(ke2)
    sqrt_e1 = jnp.sqrt(eps1)
    w1 = 1.0 - sqrt_e1
    w2 = (1.0 - eps2) * sqrt_e1
    w3 = eps2 * sqrt_e1

    N = B * _POINT_NUM
    a = st[:, :, 0, :].reshape(N, 3).T
    b = st[:, :, 1, :].reshape(N, 3).T
    c = st[:, :, 2, :].reshape(N, 3).T
    CH = 8192
    spec3 = pl.BlockSpec((3, CH), lambda i: (0, i))
    spec1 = pl.BlockSpec((1, CH), lambda i: (0, i))
    out = pl.pallas_call(
        _combine_kernel,
        grid=(N // CH,),
        in_specs=[spec3, spec3, spec3, spec1, spec1, spec1],
        out_specs=spec3,
        out_shape=jax.ShapeDtypeStruct((3, N), jnp.float32),
    )(a, b, c, w1.reshape(1, N), w2.reshape(1, N), w3.reshape(1, N))
    return out.T.reshape(B, _POINT_NUM, 3)


def kernel(vertices_batch, faces_batch):
    vb = vertices_batch.astype(jnp.float32)
    B = vb.shape[0]
    keys = jax.random.split(jax.random.key(42), B)
    trio = jax.vmap(lambda k: jax.random.split(k, 3))(keys)
    ki, ke1, ke2 = trio[:, 0], trio[:, 1], trio[:, 2]
    ki_data = jax.lax.bitcast_convert_type(
        jax.random.key_data(ki), jnp.int32)
    ke1_data = jax.lax.bitcast_convert_type(jax.random.key_data(ke1), jnp.int32)
    ke2_data = jax.lax.bitcast_convert_type(jax.random.key_data(ke2), jnp.int32)

    devs = jax.devices()
    nd = 1
    for d in (8, 4, 2):
        if len(devs) >= d and B % d == 0:
            nd = d
            break
    if nd == 1:
        return _sample_shard(vb, faces_batch, ki_data, ke1_data, ke2_data)
    mesh = jax.sharding.Mesh(np.array(devs[:nd]), ("d",))
    spec = jax.sharding.PartitionSpec("d")
    return jax.shard_map(
        _sample_shard, mesh=mesh,
        in_specs=(spec, spec, spec, spec, spec),
        out_specs=spec, check_vma=False,
    )(vb, faces_batch, ki_data, ke1_data, ke2_data)


# speedup vs baseline: 1.0374x; 1.0374x over previous
import functools

import jax
import jax.numpy as jnp
import numpy as np
from jax.experimental import pallas as pl
from jax.experimental.pallas import tpu as pltpu

_POINT_NUM = 32768
_F = 200000
_FT = 512
_FPAD = 200192
_NFT = _FPAD // _FT
_PBLK = 2048
_PT = 16

_U = np.uint32
_TINY = np.float32(1.1754943508222875e-38)


def _rotl(x, r):
    return jax.lax.shift_left(x, _U(r)) | jax.lax.shift_right_logical(x, _U(32 - r))


def _threefry_pre(k1, k2, x0, x1):
    ks2 = k1 ^ k2 ^ _U(0x1BD11BDA)
    rot = ((13, 15, 26, 6), (17, 29, 16, 24))
    sched = ((0, k2, ks2, 1), (1, ks2, k1, 2), (0, k1, k2, 3),
             (1, k2, ks2, 4), (0, ks2, k1, 5))
    for rsel, ka, kb, inc in sched:
        for r in rot[rsel]:
            x0 = x0 + x1
            x1 = _rotl(x1, r)
            x1 = x0 ^ x1
        x0 = x0 + ka
        x1 = x1 + (kb + _U(inc))
    return x0, x1


def _bits_to_unit(bits):
    fb = jax.lax.shift_right_logical(bits, _U(9)) | _U(0x3F800000)
    return jax.lax.bitcast_convert_type(fb, jnp.float32) - jnp.float32(1.0)


def _argmax_kernel(keys_ref, logits_ref, oidx_ref):
    b = pl.program_id(0)
    pblk = pl.program_id(1)
    k1 = keys_ref[b, 0].astype(jnp.uint32)
    k2 = keys_ref[b, 1].astype(jnp.uint32)

    f_lane = jax.lax.broadcasted_iota(jnp.int32, (_PT, _FT), 1)
    p_sub = jax.lax.broadcasted_iota(jnp.int32, (_PT, _FT), 0)
    p_cross = (1 << 32) // _F + 1

    def pg_body(pg, _):
        p_abs = pblk * _PBLK + pg * _PT + p_sub
        pf_lo = (p_abs * _F).astype(jnp.uint32)
        base0 = (p_abs >= p_cross).astype(jnp.uint32) + k1
        pfl = pf_lo + f_lane.astype(jnp.uint32)

        def ft_body(ft, carry):
            runval, runft = carry
            n_lo = pfl + _U(_FT) * ft.astype(jnp.uint32)
            x0 = base0 + (n_lo < pf_lo).astype(jnp.uint32)
            x1 = n_lo + k2
            o0, o1 = _threefry_pre(k1, k2, x0, x1)
            bits = o0 ^ o1
            u = jnp.maximum(_bits_to_unit(bits), _TINY)
            g = -jnp.log(-jnp.log(u))
            lg = logits_ref[0, 0, pl.ds(ft * _FT, _FT)].reshape(1, _FT)
            v = g + lg
            upd = v > runval
            return (jnp.where(upd, v, runval), jnp.where(upd, ft, runft))

        init = (jnp.full((_PT, _FT), -3.4e38, jnp.float32),
                jnp.zeros((_PT, _FT), jnp.int32))
        runval, runft = jax.lax.fori_loop(0, _NFT, ft_body, init, unroll=17)
        runidx = runft * _FT + f_lane
        rowmax = jnp.max(runval, axis=1, keepdims=True)
        masked = jnp.where(runval == rowmax, runidx, jnp.int32(0x7FFFFFFF))
        rowidx = jnp.min(masked, axis=1, keepdims=True)
        oidx_ref[0, pl.ds(pg * _PT, _PT), :] = rowidx
        return 0

    jax.lax.fori_loop(0, _PBLK // _PT, pg_body, 0)


def _categorical_argmax(keys_i32, logits_padded):
    B = logits_padded.shape[0]
    out = pl.pallas_call(
        _argmax_kernel,
        grid=(B, _POINT_NUM // _PBLK),
        in_specs=[
            pl.BlockSpec(memory_space=pltpu.SMEM),
            pl.BlockSpec((1, 1, _FPAD), lambda b, p: (b, 0, 0)),
        ],
        out_specs=pl.BlockSpec((1, _PBLK, 1), lambda b, p: (b, p, 0)),
        out_shape=jax.ShapeDtypeStruct((B, _POINT_NUM, 1), jnp.int32),
    )(keys_i32, logits_padded.reshape(B, 1, _FPAD))
    return out[:, :, 0]


def _combine_kernel(a_ref, b_ref, c_ref, w1_ref, w2_ref, w3_ref, o_ref):
    o_ref[...] = (a_ref[...] * w1_ref[...] + b_ref[...] * w2_ref[...]
                  + c_ref[...] * w3_ref[...])


def _sample_shard(vb, faces_batch, ki_data, ke1_data, ke2_data):
    B = vb.shape[0]
    ke1 = jax.random.wrap_key_data(
        jax.lax.bitcast_convert_type(ke1_data, jnp.uint32), impl="threefry2x32")
    ke2 = jax.random.wrap_key_data(
        jax.lax.bitcast_convert_type(ke2_data, jnp.uint32), impl="threefry2x32")

    def mesh_logits(vertices, faces):
        triangles = jnp.take(vertices, faces, axis=0)
        vec1 = triangles[:, 1, :] - triangles[:, 0, :]
        vec2 = triangles[:, 2, :] - triangles[:, 0, :]
        areas = jnp.linalg.norm(jnp.cross(vec1, vec2, axis=-1), axis=1) / 2.0
        return triangles, jnp.log(areas + 1e-12)

    triangles, logits = jax.vmap(mesh_logits)(vb, faces_batch)
    logits_padded = jnp.concatenate(
        [logits, jnp.full((B, _FPAD - _F), -1e30, jnp.float32)], axis=1)

    idx = _categorical_argmax(ki_data, logits_padded)

    st = jax.vmap(lambda t, i: jnp.take(t, i, axis=0))(triangles, idx)

    eps1 = jax.vmap(lambda k: jax.random.uniform(
        k, (_POINT_NUM,), dtype=jnp.float32))(ke1)
    eps2 = jax.vmap(lambda k: jax.random.uniform(
        k, (_POINT_NUM,), dtype=jnp.float32))(ke2)
    sqrt_e1 = jnp.sqrt(eps1)
    w1 = 1.0 - sqrt_e1
    w2 = (1.0 - eps2) * sqrt_e1
    w3 = eps2 * sqrt_e1

    N = B * _POINT_NUM
    a = st[:, :, 0, :].reshape(N, 3).T
    b = st[:, :, 1, :].reshape(N, 3).T
    c = st[:, :, 2, :].reshape(N, 3).T
    CH = 8192
    spec3 = pl.BlockSpec((3, CH), lambda i: (0, i))
    spec1 = pl.BlockSpec((1, CH), lambda i: (0, i))
    out = pl.pallas_call(
        _combine_kernel,
        grid=(N // CH,),
        in_specs=[spec3, spec3, spec3, spec1, spec1, spec1],
        out_specs=spec3,
        out_shape=jax.ShapeDtypeStruct((3, N), jnp.float32),
    )(a, b, c, w1.reshape(1, N), w2.reshape(1, N), w3.reshape(1, N))
    return out.T.reshape(B, _POINT_NUM, 3)


def kernel(vertices_batch, faces_batch):
    vb = vertices_batch.astype(jnp.float32)
    B = vb.shape[0]
    keys = jax.random.split(jax.random.key(42), B)
    trio = jax.vmap(lambda k: jax.random.split(k, 3))(keys)
    ki, ke1, ke2 = trio[:, 0], trio[:, 1], trio[:, 2]
    ki_data = jax.lax.bitcast_convert_type(
        jax.random.key_data(ki), jnp.int32)
    ke1_data = jax.lax.bitcast_convert_type(jax.random.key_data(ke1), jnp.int32)
    ke2_data = jax.lax.bitcast_convert_type(jax.random.key_data(ke2), jnp.int32)

    devs = jax.devices()
    nd = 1
    for d in (8, 4, 2):
        if len(devs) >= d and B % d == 0:
            nd = d
            break
    if nd == 1:
        return _sample_shard(vb, faces_batch, ki_data, ke1_data, ke2_data)
    mesh = jax.sharding.Mesh(np.array(devs[:nd]), ("d",))
    spec = jax.sharding.PartitionSpec("d")
    return jax.shard_map(
        _sample_shard, mesh=mesh,
        in_specs=(spec, spec, spec, spec, spec),
        out_specs=spec, check_vma=False,
    )(vb, faces_batch, ki_data, ke1_data, ke2_data)


# speedup vs baseline: 1.0551x; 1.0171x over previous
import functools

import jax
import jax.numpy as jnp
import numpy as np
from jax.experimental import pallas as pl
from jax.experimental.pallas import tpu as pltpu

_POINT_NUM = 32768
_F = 200000
_FT = 512
_FPAD = 200192
_NFT = _FPAD // _FT
_PBLK = 2048
_PT = 16

_U = np.uint32
_TINY = np.float32(1.1754943508222875e-38)


def _rotl(x, r):
    return jax.lax.shift_left(x, _U(r)) | jax.lax.shift_right_logical(x, _U(32 - r))


def _threefry_pre(k1, k2, x0, x1):
    ks2 = k1 ^ k2 ^ _U(0x1BD11BDA)
    rot = ((13, 15, 26, 6), (17, 29, 16, 24))
    sched = ((0, k2, ks2, 1), (1, ks2, k1, 2), (0, k1, k2, 3),
             (1, k2, ks2, 4), (0, ks2, k1, 5))
    for rsel, ka, kb, inc in sched:
        for r in rot[rsel]:
            x0 = x0 + x1
            x1 = _rotl(x1, r)
            x1 = x0 ^ x1
        x0 = x0 + ka
        x1 = x1 + (kb + _U(inc))
    return x0, x1


def _bits_to_unit(bits):
    fb = jax.lax.shift_right_logical(bits, _U(9)) | _U(0x3F800000)
    return jax.lax.bitcast_convert_type(fb, jnp.float32) - jnp.float32(1.0)


def _argmax_kernel(keys_ref, logits_ref, oidx_ref):
    b = pl.program_id(0)
    pblk = pl.program_id(1)
    k1 = keys_ref[b, 0].astype(jnp.uint32)
    k2 = keys_ref[b, 1].astype(jnp.uint32)

    f_lane = jax.lax.broadcasted_iota(jnp.int32, (_PT, _FT), 1)
    p_sub = jax.lax.broadcasted_iota(jnp.int32, (_PT, _FT), 0)
    p_cross = (1 << 32) // _F + 1

    def pg_body(pg, _):
        p_abs = pblk * _PBLK + pg * _PT + p_sub
        pf_lo = (p_abs * _F).astype(jnp.uint32)
        base0 = (p_abs >= p_cross).astype(jnp.uint32) + k1
        base0c = base0 + _U(1)
        pfl = pf_lo + f_lane.astype(jnp.uint32)

        def ft_body(ft, carry):
            runval, runft = carry
            n_lo = pfl + _U(_FT) * ft.astype(jnp.uint32)
            x0 = jnp.where(n_lo < pf_lo, base0c, base0)
            x1 = n_lo + k2
            o0, o1 = _threefry_pre(k1, k2, x0, x1)
            bits = o0 ^ o1
            u = jnp.maximum(_bits_to_unit(bits), _TINY)
            lg = logits_ref[0, 0, pl.ds(ft * _FT, _FT)].reshape(1, _FT)
            v = lg - jnp.log(-jnp.log(u))
            upd = v > runval
            return (jnp.where(upd, v, runval), jnp.where(upd, ft, runft))

        init = (jnp.full((_PT, _FT), -3.4e38, jnp.float32),
                jnp.zeros((_PT, _FT), jnp.int32))
        runval, runft = jax.lax.fori_loop(0, _NFT, ft_body, init, unroll=17)
        runidx = runft * _FT + f_lane
        rowmax = jnp.max(runval, axis=1, keepdims=True)
        masked = jnp.where(runval == rowmax, runidx, jnp.int32(0x7FFFFFFF))
        rowidx = jnp.min(masked, axis=1, keepdims=True)
        oidx_ref[0, pl.ds(pg * _PT, _PT), :] = rowidx
        return 0

    jax.lax.fori_loop(0, _PBLK // _PT, pg_body, 0)


def _categorical_argmax(keys_i32, logits_padded):
    B = logits_padded.shape[0]
    out = pl.pallas_call(
        _argmax_kernel,
        grid=(B, _POINT_NUM // _PBLK),
        in_specs=[
            pl.BlockSpec(memory_space=pltpu.SMEM),
            pl.BlockSpec((1, 1, _FPAD), lambda b, p: (b, 0, 0)),
        ],
        out_specs=pl.BlockSpec((1, _PBLK, 1), lambda b, p: (b, p, 0)),
        out_shape=jax.ShapeDtypeStruct((B, _POINT_NUM, 1), jnp.int32),
    )(keys_i32, logits_padded.reshape(B, 1, _FPAD))
    return out[:, :, 0]


def _combine_kernel(a_ref, b_ref, c_ref, w1_ref, w2_ref, w3_ref, o_ref):
    o_ref[...] = (a_ref[...] * w1_ref[...] + b_ref[...] * w2_ref[...]
                  + c_ref[...] * w3_ref[...])


def _sample_shard(vb, faces_batch, ki_data, ke1_data, ke2_data):
    B = vb.shape[0]
    ke1 = jax.random.wrap_key_data(
        jax.lax.bitcast_convert_type(ke1_data, jnp.uint32), impl="threefry2x32")
    ke2 = jax.random.wrap_key_data(
        jax.lax.bitcast_convert_type(ke2_data, jnp.uint32), impl="threefry2x32")

    def mesh_logits(vertices, faces):
        triangles = jnp.take(vertices, faces, axis=0)
        vec1 = triangles[:, 1, :] - triangles[:, 0, :]
        vec2 = triangles[:, 2, :] - triangles[:, 0, :]
        areas = jnp.linalg.norm(jnp.cross(vec1, vec2, axis=-1), axis=1) / 2.0
        return triangles, jnp.log(areas + 1e-12)

    triangles, logits = jax.vmap(mesh_logits)(vb, faces_batch)
    logits_padded = jnp.concatenate(
        [logits, jnp.full((B, _FPAD - _F), -1e30, jnp.float32)], axis=1)

    idx = _categorical_argmax(ki_data, logits_padded)

    st = jax.vmap(lambda t, i: jnp.take(t, i, axis=0))(triangles, idx)

    eps1 = jax.vmap(lambda k: jax.random.uniform(
        k, (_POINT_NUM,), dtype=jnp.float32))(ke1)
    eps2 = jax.vmap(lambda k: jax.random.uniform(
        k, (_POINT_NUM,), dtype=jnp.float32))(ke2)
    sqrt_e1 = jnp.sqrt(eps1)
    w1 = 1.0 - sqrt_e1
    w2 = (1.0 - eps2) * sqrt_e1
    w3 = eps2 * sqrt_e1

    N = B * _POINT_NUM
    a = st[:, :, 0, :].reshape(N, 3).T
    b = st[:, :, 1, :].reshape(N, 3).T
    c = st[:, :, 2, :].reshape(N, 3).T
    CH = 8192
    spec3 = pl.BlockSpec((3, CH), lambda i: (0, i))
    spec1 = pl.BlockSpec((1, CH), lambda i: (0, i))
    out = pl.pallas_call(
        _combine_kernel,
        grid=(N // CH,),
        in_specs=[spec3, spec3, spec3, spec1, spec1, spec1],
        out_specs=spec3,
        out_shape=jax.ShapeDtypeStruct((3, N), jnp.float32),
    )(a, b, c, w1.reshape(1, N), w2.reshape(1, N), w3.reshape(1, N))
    return out.T.reshape(B, _POINT_NUM, 3)


def kernel(vertices_batch, faces_batch):
    vb = vertices_batch.astype(jnp.float32)
    B = vb.shape[0]
    keys = jax.random.split(jax.random.key(42), B)
    trio = jax.vmap(lambda k: jax.random.split(k, 3))(keys)
    ki, ke1, ke2 = trio[:, 0], trio[:, 1], trio[:, 2]
    ki_data = jax.lax.bitcast_convert_type(
        jax.random.key_data(ki), jnp.int32)
    ke1_data = jax.lax.bitcast_convert_type(jax.random.key_data(ke1), jnp.int32)
    ke2_data = jax.lax.bitcast_convert_type(jax.random.key_data(ke2), jnp.int32)

    devs = jax.devices()
    nd = 1
    for d in (8, 4, 2):
        if len(devs) >= d and B % d == 0:
            nd = d
            break
    if nd == 1:
        return _sample_shard(vb, faces_batch, ki_data, ke1_data, ke2_data)
    mesh = jax.sharding.Mesh(np.array(devs[:nd]), ("d",))
    spec = jax.sharding.PartitionSpec("d")
    return jax.shard_map(
        _sample_shard, mesh=mesh,
        in_specs=(spec, spec, spec, spec, spec),
        out_specs=spec, check_vma=False,
    )(vb, faces_batch, ki_data, ke1_data, ke2_data)
